# baseline (speedup 1.0000x reference)
import functools

import jax
import jax.numpy as jnp
from jax import lax
from jax.experimental import pallas as pl
from jax.experimental.pallas import tpu as pltpu
from jax.experimental.pallas import tpu_sc as plsc

NC, NS = 2, 16
NW = NC * NS
B, L, D = 1024, 200, 128
VOCAB = 100000
ROWS_W = B // NW
TOK_W = ROWS_W * L
PE_PAD = 200
PAD_W = TOK_W + 16
_OFFS = tuple(range(0, 208, 16))

WIN_SHIFT = 12
WIN = 1 << WIN_SHIFT
NWIN = (VOCAB + WIN - 1) // WIN
SEG = 64
ARENA = TOK_W + NWIN * SEG
PT = WIN // NS


RB = 1


def _body(embW, pe_ext, x_hbm, m_hbm, out, xf, mf, eidx, pidx,
          ridx_a, tpos_a, ppos_a, pe_l, gbuf, rl, tl, cnt_v, base_v, base_s,
          win_sh, gsem, ssem):
    c_ax = lax.axis_index("c")
    s_ax = lax.axis_index("s")
    w = s_ax * NC + c_ax
    tok0 = w * TOK_W

    pltpu.sync_copy(pe_ext, pe_l)
    pltpu.sync_copy(x_hbm.at[pl.ds(tok0, TOK_W)], xf.at[pl.ds(0, TOK_W)])
    pltpu.sync_copy(m_hbm.at[pl.ds(tok0, TOK_W)], mf.at[pl.ds(0, TOK_W)])

    lane = lax.iota(jnp.int32, 16)

    def row_body(r, _):
        carry = jnp.int32(0)
        base = pl.multiple_of(r * L, 8)
        for off in _OFFS:
            last = off == 192
            src = pl.multiple_of(base + off, 8)
            m = mf[pl.ds(src, 16)]
            xx = xf[pl.ds(src, 16)]
            if last:
                m = jnp.where(lane < 8, m, 0)
            cum = plsc.cumsum(m) + carry
            pv = jnp.where(m == 1, cum - 1, PE_PAD)
            ev = xx * m
            eidx[pl.ds(src, 16)] = ev
            pidx[pl.ds(src, 16)] = pv
            if not last:
                carry = carry + jnp.sum(m)
        return 0

    lax.fori_loop(0, ROWS_W, row_body, 0)

    zero16 = lane * 0
    def adflt(i, _):
        off = pl.multiple_of(i * 16, 8)
        ridx_a[pl.ds(off, 16)] = zero16
        tpos_a[pl.ds(off, 16)] = zero16 + tok0
        ppos_a[pl.ds(off, 16)] = zero16 + PE_PAD
        return 0

    lax.fori_loop(0, ARENA // 16, adflt, 0)
    cnt_v[pl.ds(0, 16)] = zero16
    cnt_v[pl.ds(16, 16)] = zero16

    def histo(g, _):
        ev = eidx[pl.ds(pl.multiple_of(g * 16, 8), 16)]
        bid = lax.shift_right_logical(ev, WIN_SHIFT)
        rank, lastm = plsc.scan_count(bid)
        cur = plsc.load_gather(cnt_v, [bid])
        plsc.store_scatter(cnt_v, [bid], cur + rank + (1 - RB), mask=lastm)
        return 0

    lax.fori_loop(0, TOK_W // 16, histo, 0)

    c0 = cnt_v[pl.ds(0, 16)]
    c1 = cnt_v[pl.ds(16, 16)]
    r0 = ((c0 + (SEG - 1)) >> 6) << 6
    r1 = ((c1 + (SEG - 1)) >> 6) << 6
    e0 = plsc.cumsum(r0) - r0
    e1 = plsc.cumsum(r1) - r1 + jnp.sum(r0)
    cnt_v[pl.ds(0, 16)] = e0
    cnt_v[pl.ds(16, 16)] = e1
    base_v[pl.ds(0, 16)] = e0
    base_v[pl.ds(16, 16)] = e1 + jnp.where(lane == NWIN - 16, r1, 0)
    for p in range(NWIN):
        base_s[p] = e0[p] if p < 16 else e1[p - 16]
    base_s[NWIN] = (e1[NWIN - 16] if NWIN >= 16 else e0[NWIN]) + \
        (r1[NWIN - 16] if NWIN >= 16 else r0[NWIN])

    def scat(g, _):
        off = pl.multiple_of(g * 16, 8)
        ev = eidx[pl.ds(off, 16)]
        pv = pidx[pl.ds(off, 16)]
        bid = lax.shift_right_logical(ev, WIN_SHIFT)
        rank, lastm = plsc.scan_count(bid)
        cur = plsc.load_gather(cnt_v, [bid])
        tgt = cur + rank - RB
        plsc.store_scatter(ridx_a, [tgt], ev & (WIN - 1))
        plsc.store_scatter(tpos_a, [tgt], tok0 + off + lane)
        plsc.store_scatter(ppos_a, [tgt], pv)
        plsc.store_scatter(cnt_v, [bid], tgt + 1, mask=lastm)
        return 0

    lax.fori_loop(0, TOK_W // 16, scat, 0)

    def padfill(p, _):
        pvec = zero16 + p
        cur_end = plsc.load_gather(cnt_v, [pvec])
        nxt = plsc.load_gather(base_v, [pvec + 1])
        last = jnp.maximum(cur_end - 1, 0)
        rdup = plsc.load_gather(ridx_a, [last])
        tdup = plsc.load_gather(tpos_a, [last])
        pdup = plsc.load_gather(ppos_a, [last])
        for kp in range(SEG // 16):
            idxs = cur_end + kp * 16 + lane
            mskp = idxs < nxt
            plsc.store_scatter(ridx_a, [idxs], rdup, mask=mskp)
            plsc.store_scatter(tpos_a, [idxs], tdup, mask=mskp)
            plsc.store_scatter(ppos_a, [idxs], pdup, mask=mskp)
        return 0

    lax.fori_loop(0, NWIN, padfill, 0)

    def window(p, _):
        plsc.subcore_barrier()
        win_len = jnp.minimum(WIN, VOCAB - p * WIN)
        off_s = jnp.minimum(s_ax * PT, win_len - PT)

        @pl.when(s_ax * PT < win_len)
        def _():
            pltpu.sync_copy(embW.at[pl.ds(p * WIN + off_s, PT)],
                            win_sh.at[pl.ds(off_s, PT)])

        plsc.subcore_barrier()

        base_p = base_s[p]
        nblk = (base_s[p + 1] - base_p) >> 6

        def blk(j, _):
            ab = pl.multiple_of(base_p + j * SEG, 8)
            for k in range(SEG // 16):
                rl[pl.ds(k * 16, 16)] = ridx_a[pl.ds(ab + k * 16, 16)]
                tl[pl.ds(k * 16, 16)] = tpos_a[pl.ds(ab + k * 16, 16)]
            pltpu.async_copy(win_sh.at[rl], gbuf, gsem).wait()

            def peadd(q, _):
                o2 = q * 16
                pvec = ppos_a[pl.ds(ab + o2, 16)]
                pps = [pvec[j2] for j2 in range(16)]

                def pec(cc, _):
                    cof = pl.multiple_of(cc * 16, 16)
                    for j2 in range(16):
                        vec = pe_l[pps[j2], pl.ds(cof, 16)]
                        plsc.addupdate(gbuf.at[o2 + j2, pl.ds(cof, 16)], vec)
                    return 0

                lax.fori_loop(0, D // 16, pec, 0)
                return 0

            lax.fori_loop(0, SEG // 16, peadd, 0)
            pltpu.async_copy(gbuf, out.at[tl], ssem).wait()
            return 0

        lax.fori_loop(0, nblk, blk, 0)
        return 0

    lax.fori_loop(0, NWIN, window, 0)


@functools.partial(jax.jit, static_argnums=())
def kernel(embed_W, pe, x, mask):
    x = x.astype(jnp.int32).reshape(B * L)
    mask = mask.astype(jnp.int32).reshape(B * L)
    pe_ext = jnp.concatenate([pe, -embed_W[:1]], axis=0)
    mesh = plsc.VectorSubcoreMesh(core_axis_name="c", subcore_axis_name="s",
                                  num_cores=NC, num_subcores=NS)
    out = pl.kernel(
        _body,
        out_type=jax.ShapeDtypeStruct((B * L, D), jnp.float32),
        mesh=mesh,
        compiler_params=pltpu.CompilerParams(needs_layout_passes=False),
        scratch_types=[
            pltpu.VMEM((PAD_W,), jnp.int32),
            pltpu.VMEM((PAD_W,), jnp.int32),
            pltpu.VMEM((PAD_W,), jnp.int32),
            pltpu.VMEM((PAD_W,), jnp.int32),
            pltpu.VMEM((ARENA,), jnp.int32),
            pltpu.VMEM((ARENA,), jnp.int32),
            pltpu.VMEM((ARENA,), jnp.int32),
            pltpu.VMEM((L + 1, D), jnp.float32),
            pltpu.VMEM((SEG, D), jnp.float32),
            pltpu.VMEM((SEG,), jnp.int32),
            pltpu.VMEM((SEG,), jnp.int32),
            pltpu.VMEM((32,), jnp.int32),
            pltpu.VMEM((32,), jnp.int32),
            pltpu.SMEM((NWIN + 2,), jnp.int32),
            pltpu.VMEM_SHARED((WIN, D), jnp.float32),
            pltpu.SemaphoreType.DMA,
            pltpu.SemaphoreType.DMA,
        ],
    )(embed_W, pe_ext, x, mask)
    return out.reshape(B, L, D)


# speedup vs baseline: 1.0057x; 1.0057x over previous
import functools

import jax
import jax.numpy as jnp
from jax import lax
from jax.experimental import pallas as pl
from jax.experimental.pallas import tpu as pltpu
from jax.experimental.pallas import tpu_sc as plsc

NC, NS = 2, 16
NW = NC * NS
B, L, D = 1024, 200, 128
VOCAB = 100000
ROWS_W = B // NW
TOK_W = ROWS_W * L
PE_PAD = 200
PAD_W = TOK_W + 16
_OFFS = tuple(range(0, 208, 16))

WIN_SHIFT = 11
WIN = 1 << WIN_SHIFT
NWIN = (VOCAB + WIN - 1) // WIN
SEG = 64
ARENA = TOK_W + NWIN * SEG
PT = WIN // NS


RB = 1


def _body(embW, pe_ext, x_hbm, m_hbm, out, xf, mf, eidx, pidx,
          ridx_a, tpos_a, ppos_a, pe_l, gbuf, rl, tl, cnt_v, base_v, base_s,
          win_a, win_b, gsem, ssem, sem_a, sem_b):
    c_ax = lax.axis_index("c")
    s_ax = lax.axis_index("s")
    w = s_ax * NC + c_ax
    tok0 = w * TOK_W

    pltpu.sync_copy(pe_ext, pe_l)
    pltpu.sync_copy(x_hbm.at[pl.ds(tok0, TOK_W)], xf.at[pl.ds(0, TOK_W)])
    pltpu.sync_copy(m_hbm.at[pl.ds(tok0, TOK_W)], mf.at[pl.ds(0, TOK_W)])

    lane = lax.iota(jnp.int32, 16)

    def row_body(r, _):
        carry = jnp.int32(0)
        base = pl.multiple_of(r * L, 8)
        for off in _OFFS:
            last = off == 192
            src = pl.multiple_of(base + off, 8)
            m = mf[pl.ds(src, 16)]
            xx = xf[pl.ds(src, 16)]
            if last:
                m = jnp.where(lane < 8, m, 0)
            cum = plsc.cumsum(m) + carry
            pv = jnp.where(m == 1, cum - 1, PE_PAD)
            ev = xx * m
            eidx[pl.ds(src, 16)] = ev
            pidx[pl.ds(src, 16)] = pv
            if not last:
                carry = carry + jnp.sum(m)
        return 0

    lax.fori_loop(0, ROWS_W, row_body, 0)

    zero16 = lane * 0
    def adflt(i, _):
        off = pl.multiple_of(i * 16, 8)
        ridx_a[pl.ds(off, 16)] = zero16
        tpos_a[pl.ds(off, 16)] = zero16 + tok0
        ppos_a[pl.ds(off, 16)] = zero16 + PE_PAD
        return 0

    lax.fori_loop(0, ARENA // 16, adflt, 0)
    for q in range(4):
        cnt_v[pl.ds(q * 16, 16)] = zero16

    def histo(g, _):
        ev = eidx[pl.ds(pl.multiple_of(g * 16, 8), 16)]
        bid = lax.shift_right_logical(ev, WIN_SHIFT)
        rank, lastm = plsc.scan_count(bid)
        cur = plsc.load_gather(cnt_v, [bid])
        plsc.store_scatter(cnt_v, [bid], cur + rank + (1 - RB), mask=lastm)
        return 0

    lax.fori_loop(0, TOK_W // 16, histo, 0)

    cs = [cnt_v[pl.ds(q * 16, 16)] for q in range(4)]
    rs = [((c + (SEG - 1)) >> 6) << 6 for c in cs]
    es = []
    run = jnp.int32(0)
    for q in range(4):
        es.append(plsc.cumsum(rs[q]) - rs[q] + run)
        run = run + jnp.sum(rs[q])
    for q in range(4):
        cnt_v[pl.ds(q * 16, 16)] = es[q]
        bv = es[q]
        if q == NWIN // 16:
            bv = bv + jnp.where(lane == NWIN % 16, rs[q], 0)
        base_v[pl.ds(q * 16, 16)] = bv
    for p in range(NWIN):
        base_s[p] = es[p // 16][p % 16]
    base_s[NWIN] = es[NWIN // 16][NWIN % 16] + rs[NWIN // 16][NWIN % 16]

    def scat(g, _):
        off = pl.multiple_of(g * 16, 8)
        ev = eidx[pl.ds(off, 16)]
        pv = pidx[pl.ds(off, 16)]
        bid = lax.shift_right_logical(ev, WIN_SHIFT)
        rank, lastm = plsc.scan_count(bid)
        cur = plsc.load_gather(cnt_v, [bid])
        tgt = cur + rank - RB
        plsc.store_scatter(ridx_a, [tgt], ev & (WIN - 1))
        plsc.store_scatter(tpos_a, [tgt], tok0 + off + lane)
        plsc.store_scatter(ppos_a, [tgt], pv)
        plsc.store_scatter(cnt_v, [bid], tgt + 1, mask=lastm)
        return 0

    lax.fori_loop(0, TOK_W // 16, scat, 0)

    def padfill(p, _):
        pvec = zero16 + p
        cur_end = plsc.load_gather(cnt_v, [pvec])
        nxt = plsc.load_gather(base_v, [pvec + 1])
        last = jnp.maximum(cur_end - 1, 0)
        rdup = plsc.load_gather(ridx_a, [last])
        tdup = plsc.load_gather(tpos_a, [last])
        pdup = plsc.load_gather(ppos_a, [last])
        for kp in range(SEG // 16):
            idxs = cur_end + kp * 16 + lane
            mskp = idxs < nxt
            plsc.store_scatter(ridx_a, [idxs], rdup, mask=mskp)
            plsc.store_scatter(tpos_a, [idxs], tdup, mask=mskp)
            plsc.store_scatter(ppos_a, [idxs], pdup, mask=mskp)
        return 0

    lax.fori_loop(0, NWIN, padfill, 0)

    def fire_stage(q, buf, sem):
        win_len = jnp.minimum(WIN, VOCAB - q * WIN)
        off_s = jnp.minimum(s_ax * PT, win_len - PT)

        @pl.when(s_ax * PT < win_len)
        def _():
            pltpu.async_copy(embW.at[pl.ds(q * WIN + off_s, PT)],
                             buf.at[pl.ds(off_s, PT)], sem)

    def wait_stage(q, buf, sem):
        win_len = jnp.minimum(WIN, VOCAB - q * WIN)

        @pl.when(s_ax * PT < win_len)
        def _():
            pltpu.make_async_copy(embW.at[pl.ds(0, PT)],
                                  buf.at[pl.ds(0, PT)], sem).wait()

    def consume(p, wbuf):
        base_p = base_s[p]
        nblk = (base_s[p + 1] - base_p) >> 6

        def blk(j, _):
            ab = pl.multiple_of(base_p + j * SEG, 8)
            for k in range(SEG // 16):
                rl[pl.ds(k * 16, 16)] = ridx_a[pl.ds(ab + k * 16, 16)]
                tl[pl.ds(k * 16, 16)] = tpos_a[pl.ds(ab + k * 16, 16)]
            pltpu.async_copy(wbuf.at[rl], gbuf, gsem).wait()

            def peadd(q, _):
                o2 = q * 16
                pvec = ppos_a[pl.ds(ab + o2, 16)]
                pps = [pvec[j2] for j2 in range(16)]

                def pec(cc, _):
                    cof = pl.multiple_of(cc * 16, 16)
                    for j2 in range(16):
                        vec = pe_l[pps[j2], pl.ds(cof, 16)]
                        plsc.addupdate(gbuf.at[o2 + j2, pl.ds(cof, 16)], vec)
                    return 0

                lax.fori_loop(0, D // 16, pec, 0)
                return 0

            lax.fori_loop(0, SEG // 16, peadd, 0)
            pltpu.async_copy(gbuf, out.at[tl], ssem).wait()
            return 0

        lax.fori_loop(0, nblk, blk, 0)

    fire_stage(0, win_a, sem_a)

    def window(p, _):
        pm = lax.rem(p, 2)
        plsc.subcore_barrier()

        @pl.when(p + 1 < NWIN)
        def _():
            @pl.when(pm == 0)
            def _():
                fire_stage(p + 1, win_b, sem_b)

            @pl.when(pm == 1)
            def _():
                fire_stage(p + 1, win_a, sem_a)

        @pl.when(pm == 0)
        def _():
            wait_stage(p, win_a, sem_a)

        @pl.when(pm == 1)
        def _():
            wait_stage(p, win_b, sem_b)

        plsc.subcore_barrier()

        @pl.when(pm == 0)
        def _():
            consume(p, win_a)

        @pl.when(pm == 1)
        def _():
            consume(p, win_b)
        return 0

    lax.fori_loop(0, NWIN, window, 0)


@functools.partial(jax.jit, static_argnums=())
def kernel(embed_W, pe, x, mask):
    x = x.astype(jnp.int32).reshape(B * L)
    mask = mask.astype(jnp.int32).reshape(B * L)
    pe_ext = jnp.concatenate([pe, -embed_W[:1]], axis=0)
    mesh = plsc.VectorSubcoreMesh(core_axis_name="c", subcore_axis_name="s",
                                  num_cores=NC, num_subcores=NS)
    out = pl.kernel(
        _body,
        out_type=jax.ShapeDtypeStruct((B * L, D), jnp.float32),
        mesh=mesh,
        compiler_params=pltpu.CompilerParams(needs_layout_passes=False),
        scratch_types=[
            pltpu.VMEM((PAD_W,), jnp.int32),
            pltpu.VMEM((PAD_W,), jnp.int32),
            pltpu.VMEM((PAD_W,), jnp.int32),
            pltpu.VMEM((PAD_W,), jnp.int32),
            pltpu.VMEM((ARENA,), jnp.int32),
            pltpu.VMEM((ARENA,), jnp.int32),
            pltpu.VMEM((ARENA,), jnp.int32),
            pltpu.VMEM((L + 1, D), jnp.float32),
            pltpu.VMEM((SEG, D), jnp.float32),
            pltpu.VMEM((SEG,), jnp.int32),
            pltpu.VMEM((SEG,), jnp.int32),
            pltpu.VMEM((64,), jnp.int32),
            pltpu.VMEM((64,), jnp.int32),
            pltpu.SMEM((NWIN + 2,), jnp.int32),
            pltpu.VMEM_SHARED((WIN, D), jnp.float32),
            pltpu.VMEM_SHARED((WIN, D), jnp.float32),
            pltpu.SemaphoreType.DMA,
            pltpu.SemaphoreType.DMA,
            pltpu.SemaphoreType.DMA,
            pltpu.SemaphoreType.DMA,
        ],
    )(embed_W, pe_ext, x, mask)
    return out.reshape(B, L, D)
